# baseline (speedup 1.0000x reference)
import functools

import jax
import jax.numpy as jnp
from jax import lax
from jax.experimental import pallas as pl
from jax.experimental.pallas import tpu as pltpu
from jax.experimental.pallas import tpu_sc as plsc

N, E, D = 10000, 320000, 128
NC, NS, L = 2, 16, 16
NW = NC * NS
EPT = E // NW
C1 = 2000
NC1 = EPT // C1
K = 80
EPTP = 10240
NK = EPTP // K
NP = 10112
RPT = NP // NS

_MESH = plsc.VectorSubcoreMesh(
    core_axis_name="c", subcore_axis_name="s", num_cores=NC, num_subcores=NS)
_SC_PARAMS = pltpu.CompilerParams(needs_layout_passes=False)


def _pre_body(x_r, wg_r, asr_r, adr_r, wc_r, xw_r, as_r, ad_r, xg_r):
    xx = x_r[...]
    xw = jnp.dot(xx, wg_r[...], preferred_element_type=jnp.float32)
    xw_r[...] = xw
    as_r[...] = jnp.sum(xw * asr_r[...], axis=1, keepdims=True)
    ad_r[...] = jnp.sum(xw * adr_r[...], axis=1, keepdims=True)
    xg_r[...] = jnp.dot(xx, wc_r[...], preferred_element_type=jnp.float32)


def _tc_pre(x, W_gat, att_src, att_dst, W_gcn):
    return pl.pallas_call(
        _pre_body,
        out_shape=(
            jax.ShapeDtypeStruct((N, D), jnp.float32),
            jax.ShapeDtypeStruct((N, 1), jnp.float32),
            jax.ShapeDtypeStruct((N, 1), jnp.float32),
            jax.ShapeDtypeStruct((N, D), jnp.float32),
        ),
    )(x, W_gat, att_src.reshape(1, D), att_dst.reshape(1, D), W_gcn)


def _s1_body(src_h, dst_h, as_h, ad_h, p_h, denp_h, degp_h,
             asv, adv, den, deg, sidx, didx, pbuf):
    c = lax.axis_index("c")
    s = lax.axis_index("s")
    wid = c * NS + s
    base = wid * EPT
    pltpu.sync_copy(as_h, asv)
    pltpu.sync_copy(ad_h, adv)

    def zero(i, carry):
        den[pl.ds(i * L, L)] = jnp.zeros((L,), jnp.float32)
        deg[pl.ds(i * L, L)] = jnp.zeros((L,), jnp.float32)
        return carry

    lax.fori_loop(0, N // L, zero, 0)

    for cc in range(NC1):
        off = pl.multiple_of(base + cc * C1, 8)
        pltpu.sync_copy(src_h.at[pl.ds(off, C1)], sidx)
        pltpu.sync_copy(dst_h.at[pl.ds(off, C1)], didx)

        def work(j, carry):
            si = sidx[pl.ds(j * L, L)]
            di = didx[pl.ds(j * L, L)]
            av = plsc.load_gather(asv, [si])
            dv = plsc.load_gather(adv, [di])
            e = av + dv
            e = jnp.where(e > 0, e, 0.2 * e)
            p = jnp.exp(e)
            pbuf[pl.ds(j * L, L)] = p
            plsc.addupdate_scatter(den, [di], p)
            plsc.addupdate_scatter(deg, [di], jnp.full((L,), 1.0, jnp.float32))
            return carry

        lax.fori_loop(0, C1 // L, work, 0)
        pltpu.sync_copy(pbuf, p_h.at[pl.ds(off, C1)])

    pltpu.sync_copy(den, denp_h.at[pl.ds(wid * N, N)])
    pltpu.sync_copy(deg, degp_h.at[pl.ds(wid * N, N)])


def _sc_scalar(src, dst, a_s, a_d):
    return pl.kernel(
        _s1_body,
        out_type=(
            jax.ShapeDtypeStruct((E,), jnp.float32),
            jax.ShapeDtypeStruct((NW * N,), jnp.float32),
            jax.ShapeDtypeStruct((NW * N,), jnp.float32),
        ),
        mesh=_MESH,
        compiler_params=_SC_PARAMS,
        scratch_types=[
            pltpu.VMEM((N,), jnp.float32),
            pltpu.VMEM((N,), jnp.float32),
            pltpu.VMEM((N,), jnp.float32),
            pltpu.VMEM((N,), jnp.float32),
            pltpu.VMEM((C1,), jnp.int32),
            pltpu.VMEM((C1,), jnp.int32),
            pltpu.VMEM((C1,), jnp.float32),
        ],
    )(src, dst, a_s, a_d)


def _mid_body(denp_r, degp_r, as_r, ad_r, xg_r,
              xgd_r, den_r, dinv_r, ps_r):
    den_e = jnp.sum(denp_r[...], axis=1, keepdims=True)
    deg = jnp.sum(degp_r[...], axis=1, keepdims=True) + 1.0
    e = as_r[...] + ad_r[...]
    e = jnp.where(e > 0, e, 0.2 * e)
    ps = jnp.exp(e)
    ps_r[...] = ps
    den_r[...] = den_e + ps
    dinv = lax.rsqrt(deg)
    dinv_r[...] = dinv
    xgd_r[...] = xg_r[...] * dinv


def _tc_mid(denpT, degpT, a_s, a_d, xg):
    return pl.pallas_call(
        _mid_body,
        out_shape=(
            jax.ShapeDtypeStruct((N, D), jnp.float32),
            jax.ShapeDtypeStruct((N, 1), jnp.float32),
            jax.ShapeDtypeStruct((N, 1), jnp.float32),
            jax.ShapeDtypeStruct((N, 1), jnp.float32),
        ),
    )(denpT, degpT, a_s, a_d, xg)


def _s2_body(src_h, dst_h, p_h, xw_h, x_h, xgd_h, z_h,
             gat_h, gin_h, gcn_h,
             acc, sidxa, pca, didx0, didx1, didx2, didx3, rows0, rows1,
             sem0, sem1, msem0, msem1, msem2, msem3, ssem0, ssem1):
    c = lax.axis_index("c")
    s = lax.axis_index("s")
    wid = c * NS + s
    srow = s * RPT
    ebase = wid * EPTP
    sems = (sem0, sem1)
    ssems = (ssem0, ssem1)
    bufs = (rows0, rows1)
    didxs = (didx0, didx1, didx2, didx3)
    msems = (msem0, msem1, msem2, msem3)

    pltpu.sync_copy(src_h.at[pl.ds(ebase, EPTP)], sidxa)
    pltpu.sync_copy(p_h.at[pl.ds(ebase, EPTP)], pca)

    def didx_start(cc, m):
        off = pl.multiple_of(ebase + cc * K, 8)
        pltpu.make_async_copy(dst_h.at[pl.ds(off, K)], didxs[m],
                              msems[m]).start()

    def didx_wait(cc, m):
        off = pl.multiple_of(ebase + cc * K, 8)
        pltpu.make_async_copy(dst_h.at[pl.ds(off, K)], didxs[m],
                              msems[m]).wait()

    def scat_wait(nb, pm):
        pltpu.make_async_copy(bufs[nb], acc.at[didxs[pm]], ssems[nb]).wait()

    def sidx_at(cc):
        return sidxa.at[pl.ds(pl.multiple_of(cc * K, 8), K)]

    for tab, out_h, scaled in ((xw_h, gat_h, True),
                               (x_h, gin_h, False),
                               (xgd_h, gcn_h, False)):
        pltpu.sync_copy(z_h, acc.at[pl.ds(srow, RPT)])
        plsc.subcore_barrier()

        for m in range(4):
            didx_start(m, m)
        pltpu.make_async_copy(tab.at[sidx_at(0)], bufs[0], sems[0]).start()

        def quad(i, carry):
            for b4 in range(4):
                cc = 4 * i + b4
                b = b4 % 2
                nb = 1 - b
                m = b4
                pm = (b4 + 3) % 4
                rows = bufs[b]
                pltpu.make_async_copy(tab.at[sidx_at(cc)], rows,
                                      sems[b]).wait()
                if b4 == 0:
                    @pl.when(i >= 1)
                    def _():
                        scat_wait(nb, pm)
                        didx_start(cc + 3, pm)
                else:
                    scat_wait(nb, pm)

                    @pl.when(cc + 3 < NK)
                    def _():
                        didx_start(cc + 3, pm)
                if b4 == 3:
                    @pl.when(cc + 1 < NK)
                    def _():
                        pltpu.make_async_copy(tab.at[sidx_at(cc + 1)],
                                              bufs[nb], sems[nb]).start()
                else:
                    pltpu.make_async_copy(tab.at[sidx_at(cc + 1)],
                                          bufs[nb], sems[nb]).start()
                if scaled:
                    def scale(e, cy):
                        psp = plsc.load_gather(
                            pca, [lax.broadcast(cc * K + e, (L,))])
                        for j in range(D // L):
                            rows[e, pl.ds(j * L, L)] = (
                                rows[e, pl.ds(j * L, L)] * psp)
                        return cy

                    lax.fori_loop(0, K, scale, 0, unroll=4)
                didx_wait(cc, m)
                pltpu.make_async_copy(rows, acc.at[didxs[m]],
                                      ssems[b]).start(add=True)
            return carry

        lax.fori_loop(0, NK // 4, quad, 0)
        scat_wait(1, 3)
        plsc.subcore_barrier()
        pltpu.sync_copy(acc.at[pl.ds(srow, RPT)], out_h.at[c, pl.ds(srow, RPT)])
        plsc.subcore_barrier()


def _sc_rows(src, dst, p, xw, x, xgd, zeros):
    return pl.kernel(
        _s2_body,
        out_type=(
            jax.ShapeDtypeStruct((NC, NP, D), jnp.float32),
            jax.ShapeDtypeStruct((NC, NP, D), jnp.float32),
            jax.ShapeDtypeStruct((NC, NP, D), jnp.float32),
        ),
        mesh=_MESH,
        compiler_params=_SC_PARAMS,
        scratch_types=[
            pltpu.VMEM_SHARED((NP, D), jnp.float32),
            pltpu.VMEM((EPTP,), jnp.int32),
            pltpu.VMEM((EPTP,), jnp.float32),
            pltpu.VMEM((K,), jnp.int32),
            pltpu.VMEM((K,), jnp.int32),
            pltpu.VMEM((K,), jnp.int32),
            pltpu.VMEM((K,), jnp.int32),
            pltpu.VMEM((K, D), jnp.float32),
            pltpu.VMEM((K, D), jnp.float32),
            pltpu.SemaphoreType.DMA,
            pltpu.SemaphoreType.DMA,
            pltpu.SemaphoreType.DMA,
            pltpu.SemaphoreType.DMA,
            pltpu.SemaphoreType.DMA,
            pltpu.SemaphoreType.DMA,
            pltpu.SemaphoreType.DMA,
            pltpu.SemaphoreType.DMA,
        ],
    )(src, dst, p, xw, x, xgd, zeros)


def _head1_body(gat_r, gin_r, gcn_r, xw_r, x_r, xg_r, ps_r, den_r, dinv_r,
                bg_r, w1_r, b1_r, w2_r, b2_r, bc_r,
                h_r, sums_r):
    ga = gat_r[0] + gat_r[1]
    xgat = (ga + ps_r[...] * xw_r[...]) / den_r[...] + bg_r[...]
    gi = gin_r[0] + gin_r[1]
    hh = x_r[...] + gi
    h1 = jnp.maximum(
        jnp.dot(hh, w1_r[...], preferred_element_type=jnp.float32) + b1_r[...],
        0.0)
    xgin = jnp.dot(h1, w2_r[...], preferred_element_type=jnp.float32) + b2_r[...]
    gc = gcn_r[0] + gcn_r[1]
    dv = dinv_r[...]
    xgcn = dv * gc + (dv * dv) * xg_r[...] + bc_r[...]
    hb = jnp.concatenate([xgat, xgin, xgcn], axis=1)
    h_r[...] = hb

    @pl.when(pl.program_id(0) == 0)
    def _():
        sums_r[...] = jnp.zeros_like(sums_r)

    s1 = jnp.sum(hb, axis=0, keepdims=True)
    s2 = jnp.sum(hb * hb, axis=0, keepdims=True)
    sums_r[...] += jnp.concatenate([s1, s2], axis=0)


def _tc_head1(gat, gin, gcn, xw, x, xg, ps, den, dinv,
              b_gat, gin_W1, gin_b1, gin_W2, gin_b2, b_gcn):
    B = 2000
    G = N // B
    full = lambda shape: pl.BlockSpec(shape, lambda i: (0,) * len(shape))
    row3 = pl.BlockSpec((NC, B, D), lambda i: (0, i, 0))
    row2 = pl.BlockSpec((B, D), lambda i: (i, 0))
    col = pl.BlockSpec((B, 1), lambda i: (i, 0))
    return pl.pallas_call(
        _head1_body,
        grid=(G,),
        in_specs=[
            row3, row3, row3, row2, row2, row2, col, col, col,
            full((1, D)), full((D, D)), full((1, D)), full((D, D)),
            full((1, D)), full((1, D)),
        ],
        out_specs=(
            pl.BlockSpec((B, 3 * D), lambda i: (i, 0)),
            pl.BlockSpec((2, 3 * D), lambda i: (0, 0)),
        ),
        out_shape=(
            jax.ShapeDtypeStruct((N, 3 * D), jnp.float32),
            jax.ShapeDtypeStruct((2, 3 * D), jnp.float32),
        ),
    )(gat, gin, gcn, xw, x, xg, ps, den, dinv,
      b_gat.reshape(1, D), gin_W1, gin_b1.reshape(1, D), gin_W2,
      gin_b2.reshape(1, D), b_gcn.reshape(1, D))


def _head2_body(h_r, sums_r, gam_r, bet_r, wl_r, bl_r, w2_r, b2_r, out_r):
    s = sums_r[...]
    mu = s[0:1] / float(N)
    var = s[1:2] / float(N) - mu * mu
    scale = gam_r[...] * lax.rsqrt(var + 1e-5)
    shift = bet_r[...] - mu * scale
    hb = jnp.maximum(h_r[...] * scale + shift, 0.0)
    h2 = jnp.maximum(
        jnp.dot(hb, wl_r[...], preferred_element_type=jnp.float32) + bl_r[...],
        0.0)
    out_r[...] = jnp.dot(h2, w2_r[...], preferred_element_type=jnp.float32) + b2_r[...]


def _tc_head2(h, sums, bn_gamma, bn_beta, W_lin, b_lin, W_lin2, b_lin2):
    B = 2000
    G = N // B
    full = lambda shape: pl.BlockSpec(shape, lambda i: (0,) * len(shape))
    return pl.pallas_call(
        _head2_body,
        grid=(G,),
        in_specs=[
            pl.BlockSpec((B, 3 * D), lambda i: (i, 0)),
            full((2, 3 * D)),
            full((1, 3 * D)), full((1, 3 * D)),
            full((3 * D, D)), full((1, D)),
            full((D, D)), full((1, D)),
        ],
        out_specs=pl.BlockSpec((B, D), lambda i: (i, 0)),
        out_shape=jax.ShapeDtypeStruct((N, D), jnp.float32),
    )(h, sums, bn_gamma.reshape(1, 3 * D), bn_beta.reshape(1, 3 * D),
      W_lin, b_lin.reshape(1, D), W_lin2, b_lin2.reshape(1, D))


def kernel(x, W_gat, att_src, att_dst, b_gat, gin_W1, gin_b1, gin_W2, gin_b2,
           W_gcn, b_gcn, bn_gamma, bn_beta, W_lin, b_lin, W_lin2, b_lin2,
           edge_index):
    src = edge_index[0]
    dst = edge_index[1]

    xw, a_s, a_d, xg = _tc_pre(x, W_gat, att_src, att_dst, W_gcn)

    p, denp, degp = _sc_scalar(src, dst,
                               a_s.reshape(N), a_d.reshape(N))

    xgd, den, dinv, ps = _tc_mid(denp.reshape(NW, N).T,
                                 degp.reshape(NW, N).T, a_s, a_d, xg)

    pad = ((0, 0), (0, EPTP - EPT))
    src_p = jnp.pad(src.reshape(NW, EPT), pad).reshape(NW * EPTP)
    dst_p = jnp.pad(dst.reshape(NW, EPT), pad,
                    constant_values=NP - 1).reshape(NW * EPTP)
    p_p = jnp.pad(p.reshape(NW, EPT), pad).reshape(NW * EPTP)
    zeros = jnp.zeros((RPT, D), jnp.float32)
    gat, gin, gcn = _sc_rows(src_p, dst_p, p_p, xw, x, xgd, zeros)

    h, sums = _tc_head1(gat, gin, gcn, xw, x, xg, ps, den, dinv,
                        b_gat, gin_W1, gin_b1, gin_W2, gin_b2, b_gcn)

    return _tc_head2(h, sums, bn_gamma, bn_beta, W_lin, b_lin, W_lin2, b_lin2)


# speedup vs baseline: 1.7251x; 1.7251x over previous
import functools

import jax
import jax.numpy as jnp
from jax import lax
from jax.experimental import pallas as pl
from jax.experimental.pallas import tpu as pltpu
from jax.experimental.pallas import tpu_sc as plsc

N, E, D = 10000, 320000, 128
NC, NS, L = 2, 16, 16
NW = NC * NS
EPT = E // NW
C1 = 2000
NC1 = EPT // C1
K = 112
EPTP = 10080
NK = EPTP // K
NP = 10112
RPT = NP // NS

_MESH = plsc.VectorSubcoreMesh(
    core_axis_name="c", subcore_axis_name="s", num_cores=NC, num_subcores=NS)
_SC_PARAMS = pltpu.CompilerParams(needs_layout_passes=False)


def _pre_body(x_r, wg_r, asr_r, adr_r, wc_r, xw_r, as_r, ad_r, xg_r):
    xx = x_r[...]
    xw = jnp.dot(xx, wg_r[...], preferred_element_type=jnp.float32)
    xw_r[...] = xw
    as_r[...] = jnp.sum(xw * asr_r[...], axis=1, keepdims=True)
    ad_r[...] = jnp.sum(xw * adr_r[...], axis=1, keepdims=True)
    xg_r[...] = jnp.dot(xx, wc_r[...], preferred_element_type=jnp.float32)


def _tc_pre(x, W_gat, att_src, att_dst, W_gcn):
    return pl.pallas_call(
        _pre_body,
        out_shape=(
            jax.ShapeDtypeStruct((N, D), jnp.float32),
            jax.ShapeDtypeStruct((N, 1), jnp.float32),
            jax.ShapeDtypeStruct((N, 1), jnp.float32),
            jax.ShapeDtypeStruct((N, D), jnp.float32),
        ),
    )(x, W_gat, att_src.reshape(1, D), att_dst.reshape(1, D), W_gcn)


def _s1_body(src_h, dst_h, as_h, ad_h, p_h, denp_h, degp_h,
             asv, adv, den, deg, sidx, didx, pbuf):
    c = lax.axis_index("c")
    s = lax.axis_index("s")
    wid = c * NS + s
    base = wid * EPT
    pltpu.sync_copy(as_h, asv)
    pltpu.sync_copy(ad_h, adv)

    def zero(i, carry):
        den[pl.ds(i * L, L)] = jnp.zeros((L,), jnp.float32)
        deg[pl.ds(i * L, L)] = jnp.zeros((L,), jnp.float32)
        return carry

    lax.fori_loop(0, N // L, zero, 0)

    for cc in range(NC1):
        off = pl.multiple_of(base + cc * C1, 8)
        pltpu.sync_copy(src_h.at[pl.ds(off, C1)], sidx)
        pltpu.sync_copy(dst_h.at[pl.ds(off, C1)], didx)

        def work(j, carry):
            si = sidx[pl.ds(j * L, L)]
            di = didx[pl.ds(j * L, L)]
            av = plsc.load_gather(asv, [si])
            dv = plsc.load_gather(adv, [di])
            e = av + dv
            e = jnp.where(e > 0, e, 0.2 * e)
            p = jnp.exp(e)
            pbuf[pl.ds(j * L, L)] = p
            plsc.addupdate_scatter(den, [di], p)
            plsc.addupdate_scatter(deg, [di], jnp.full((L,), 1.0, jnp.float32))
            return carry

        lax.fori_loop(0, C1 // L, work, 0)
        pltpu.sync_copy(pbuf, p_h.at[pl.ds(off, C1)])

    pltpu.sync_copy(den, denp_h.at[pl.ds(wid * N, N)])
    pltpu.sync_copy(deg, degp_h.at[pl.ds(wid * N, N)])


def _sc_scalar(src, dst, a_s, a_d):
    return pl.kernel(
        _s1_body,
        out_type=(
            jax.ShapeDtypeStruct((E,), jnp.float32),
            jax.ShapeDtypeStruct((NW * N,), jnp.float32),
            jax.ShapeDtypeStruct((NW * N,), jnp.float32),
        ),
        mesh=_MESH,
        compiler_params=_SC_PARAMS,
        scratch_types=[
            pltpu.VMEM((N,), jnp.float32),
            pltpu.VMEM((N,), jnp.float32),
            pltpu.VMEM((N,), jnp.float32),
            pltpu.VMEM((N,), jnp.float32),
            pltpu.VMEM((C1,), jnp.int32),
            pltpu.VMEM((C1,), jnp.int32),
            pltpu.VMEM((C1,), jnp.float32),
        ],
    )(src, dst, a_s, a_d)


def _mid_body(denp_r, degp_r, as_r, ad_r, xg_r,
              xgd_r, den_r, dinv_r, ps_r):
    den_e = jnp.sum(denp_r[...], axis=1, keepdims=True)
    deg = jnp.sum(degp_r[...], axis=1, keepdims=True) + 1.0
    e = as_r[...] + ad_r[...]
    e = jnp.where(e > 0, e, 0.2 * e)
    ps = jnp.exp(e)
    ps_r[...] = ps
    den_r[...] = den_e + ps
    dinv = lax.rsqrt(deg)
    dinv_r[...] = dinv
    xgd_r[...] = xg_r[...] * dinv


def _tc_mid(denpT, degpT, a_s, a_d, xg):
    return pl.pallas_call(
        _mid_body,
        out_shape=(
            jax.ShapeDtypeStruct((N, D), jnp.float32),
            jax.ShapeDtypeStruct((N, 1), jnp.float32),
            jax.ShapeDtypeStruct((N, 1), jnp.float32),
            jax.ShapeDtypeStruct((N, 1), jnp.float32),
        ),
    )(denpT, degpT, a_s, a_d, xg)


def _s2_body(src_h, dst_h, p_h, xw_h, x_h, xgd_h, z_h,
             gat_h, gin_h, gcn_h,
             acc, sidxa, pca, didx0, didx1, rows0, rows1,
             sem0, sem1, msem0, msem1):
    c = lax.axis_index("c")
    s = lax.axis_index("s")
    wid = c * NS + s
    srow = s * RPT
    ebase = wid * EPTP
    sems = (sem0, sem1)
    bufs = (rows0, rows1)
    didxs = (didx0, didx1)
    msems = (msem0, msem1)

    pltpu.sync_copy(src_h.at[pl.ds(ebase, EPTP)], sidxa)
    pltpu.sync_copy(p_h.at[pl.ds(ebase, EPTP)], pca)

    def didx_start(cc, m):
        off = pl.multiple_of(ebase + cc * K, 8)
        pltpu.make_async_copy(dst_h.at[pl.ds(off, K)], didxs[m],
                              msems[m]).start()

    def didx_wait(cc, m):
        off = pl.multiple_of(ebase + cc * K, 8)
        pltpu.make_async_copy(dst_h.at[pl.ds(off, K)], didxs[m],
                              msems[m]).wait()

    def sidx_at(cc):
        return sidxa.at[pl.ds(pl.multiple_of(cc * K, 8), K)]

    for tab, out_h, scaled in ((xw_h, gat_h, True),
                               (x_h, gin_h, False),
                               (xgd_h, gcn_h, False)):
        pltpu.sync_copy(z_h, acc.at[pl.ds(srow, RPT)])
        plsc.subcore_barrier()

        didx_start(0, 0)
        didx_start(1, 1)
        pltpu.make_async_copy(tab.at[sidx_at(0)], bufs[0], sems[0]).start()

        def pair(i, carry):
            for b in range(2):
                cc = 2 * i + b
                nb = 1 - b
                rows = bufs[b]
                pltpu.make_async_copy(tab.at[sidx_at(cc)], rows,
                                      sems[b]).wait()

                @pl.when(cc + 1 < NK)
                def _():
                    pltpu.make_async_copy(tab.at[sidx_at(cc + 1)], bufs[nb],
                                          sems[nb]).start()
                if scaled:
                    def scale(e, cy):
                        psp = plsc.load_gather(
                            pca, [lax.broadcast(cc * K + e, (L,))])
                        for j in range(D // L):
                            rows[e, pl.ds(j * L, L)] = (
                                rows[e, pl.ds(j * L, L)] * psp)
                        return cy

                    lax.fori_loop(0, K, scale, 0, unroll=4)
                didx_wait(cc, b)
                pltpu.sync_copy(rows, acc.at[didxs[b]], add=True)

                @pl.when(cc + 2 < NK)
                def _():
                    didx_start(cc + 2, b)
            return carry

        lax.fori_loop(0, NK // 2, pair, 0)
        plsc.subcore_barrier()
        pltpu.sync_copy(acc.at[pl.ds(srow, RPT)], out_h.at[c, pl.ds(srow, RPT)])
        plsc.subcore_barrier()


def _sc_rows(src, dst, p, xw, x, xgd, zeros):
    return pl.kernel(
        _s2_body,
        out_type=(
            jax.ShapeDtypeStruct((NC, NP, D), jnp.float32),
            jax.ShapeDtypeStruct((NC, NP, D), jnp.float32),
            jax.ShapeDtypeStruct((NC, NP, D), jnp.float32),
        ),
        mesh=_MESH,
        compiler_params=_SC_PARAMS,
        scratch_types=[
            pltpu.VMEM_SHARED((NP, D), jnp.float32),
            pltpu.VMEM((EPTP,), jnp.int32),
            pltpu.VMEM((EPTP,), jnp.float32),
            pltpu.VMEM((K,), jnp.int32),
            pltpu.VMEM((K,), jnp.int32),
            pltpu.VMEM((K, D), jnp.float32),
            pltpu.VMEM((K, D), jnp.float32),
            pltpu.SemaphoreType.DMA,
            pltpu.SemaphoreType.DMA,
            pltpu.SemaphoreType.DMA,
            pltpu.SemaphoreType.DMA,
        ],
    )(src, dst, p, xw, x, xgd, zeros)


def _head1_body(gat_r, gin_r, gcn_r, xw_r, x_r, xg_r, ps_r, den_r, dinv_r,
                bg_r, w1_r, b1_r, w2_r, b2_r, bc_r,
                h_r, sums_r):
    ga = gat_r[0] + gat_r[1]
    xgat = (ga + ps_r[...] * xw_r[...]) / den_r[...] + bg_r[...]
    gi = gin_r[0] + gin_r[1]
    hh = x_r[...] + gi
    h1 = jnp.maximum(
        jnp.dot(hh, w1_r[...], preferred_element_type=jnp.float32) + b1_r[...],
        0.0)
    xgin = jnp.dot(h1, w2_r[...], preferred_element_type=jnp.float32) + b2_r[...]
    gc = gcn_r[0] + gcn_r[1]
    dv = dinv_r[...]
    xgcn = dv * gc + (dv * dv) * xg_r[...] + bc_r[...]
    hb = jnp.concatenate([xgat, xgin, xgcn], axis=1)
    h_r[...] = hb

    @pl.when(pl.program_id(0) == 0)
    def _():
        sums_r[...] = jnp.zeros_like(sums_r)

    s1 = jnp.sum(hb, axis=0, keepdims=True)
    s2 = jnp.sum(hb * hb, axis=0, keepdims=True)
    sums_r[...] += jnp.concatenate([s1, s2], axis=0)


def _tc_head1(gat, gin, gcn, xw, x, xg, ps, den, dinv,
              b_gat, gin_W1, gin_b1, gin_W2, gin_b2, b_gcn):
    B = 2000
    G = N // B
    full = lambda shape: pl.BlockSpec(shape, lambda i: (0,) * len(shape))
    row3 = pl.BlockSpec((NC, B, D), lambda i: (0, i, 0))
    row2 = pl.BlockSpec((B, D), lambda i: (i, 0))
    col = pl.BlockSpec((B, 1), lambda i: (i, 0))
    return pl.pallas_call(
        _head1_body,
        grid=(G,),
        in_specs=[
            row3, row3, row3, row2, row2, row2, col, col, col,
            full((1, D)), full((D, D)), full((1, D)), full((D, D)),
            full((1, D)), full((1, D)),
        ],
        out_specs=(
            pl.BlockSpec((B, 3 * D), lambda i: (i, 0)),
            pl.BlockSpec((2, 3 * D), lambda i: (0, 0)),
        ),
        out_shape=(
            jax.ShapeDtypeStruct((N, 3 * D), jnp.float32),
            jax.ShapeDtypeStruct((2, 3 * D), jnp.float32),
        ),
    )(gat, gin, gcn, xw, x, xg, ps, den, dinv,
      b_gat.reshape(1, D), gin_W1, gin_b1.reshape(1, D), gin_W2,
      gin_b2.reshape(1, D), b_gcn.reshape(1, D))


def _head2_body(h_r, sums_r, gam_r, bet_r, wl_r, bl_r, w2_r, b2_r, out_r):
    s = sums_r[...]
    mu = s[0:1] / float(N)
    var = s[1:2] / float(N) - mu * mu
    scale = gam_r[...] * lax.rsqrt(var + 1e-5)
    shift = bet_r[...] - mu * scale
    hb = jnp.maximum(h_r[...] * scale + shift, 0.0)
    h2 = jnp.maximum(
        jnp.dot(hb, wl_r[...], preferred_element_type=jnp.float32) + bl_r[...],
        0.0)
    out_r[...] = jnp.dot(h2, w2_r[...], preferred_element_type=jnp.float32) + b2_r[...]


def _tc_head2(h, sums, bn_gamma, bn_beta, W_lin, b_lin, W_lin2, b_lin2):
    B = 2000
    G = N // B
    full = lambda shape: pl.BlockSpec(shape, lambda i: (0,) * len(shape))
    return pl.pallas_call(
        _head2_body,
        grid=(G,),
        in_specs=[
            pl.BlockSpec((B, 3 * D), lambda i: (i, 0)),
            full((2, 3 * D)),
            full((1, 3 * D)), full((1, 3 * D)),
            full((3 * D, D)), full((1, D)),
            full((D, D)), full((1, D)),
        ],
        out_specs=pl.BlockSpec((B, D), lambda i: (i, 0)),
        out_shape=jax.ShapeDtypeStruct((N, D), jnp.float32),
    )(h, sums, bn_gamma.reshape(1, 3 * D), bn_beta.reshape(1, 3 * D),
      W_lin, b_lin.reshape(1, D), W_lin2, b_lin2.reshape(1, D))


def kernel(x, W_gat, att_src, att_dst, b_gat, gin_W1, gin_b1, gin_W2, gin_b2,
           W_gcn, b_gcn, bn_gamma, bn_beta, W_lin, b_lin, W_lin2, b_lin2,
           edge_index):
    src = edge_index[0]
    dst = edge_index[1]

    xw, a_s, a_d, xg = _tc_pre(x, W_gat, att_src, att_dst, W_gcn)

    p, denp, degp = _sc_scalar(src, dst,
                               a_s.reshape(N), a_d.reshape(N))

    xgd, den, dinv, ps = _tc_mid(denp.reshape(NW, N).T,
                                 degp.reshape(NW, N).T, a_s, a_d, xg)

    pad = ((0, 0), (0, EPTP - EPT))
    src_p = jnp.pad(src.reshape(NW, EPT), pad).reshape(NW * EPTP)
    dst_p = jnp.pad(dst.reshape(NW, EPT), pad,
                    constant_values=NP - 1).reshape(NW * EPTP)
    p_p = jnp.pad(p.reshape(NW, EPT), pad).reshape(NW * EPTP)
    zeros = jnp.zeros((RPT, D), jnp.float32)
    gat, gin, gcn = _sc_rows(src_p, dst_p, p_p, xw, x, xgd, zeros)

    h, sums = _tc_head1(gat, gin, gcn, xw, x, xg, ps, den, dinv,
                        b_gat, gin_W1, gin_b1, gin_W2, gin_b2, b_gcn)

    return _tc_head2(h, sums, bn_gamma, bn_beta, W_lin, b_lin, W_lin2, b_lin2)


# speedup vs baseline: 1.7280x; 1.0017x over previous
import functools

import jax
import jax.numpy as jnp
from jax import lax
from jax.experimental import pallas as pl
from jax.experimental.pallas import tpu as pltpu
from jax.experimental.pallas import tpu_sc as plsc

N, E, D = 10000, 320000, 128
NC, NS, L = 2, 16, 16
NW = NC * NS
EPT = E // NW
C1 = 2000
NC1 = EPT // C1
K = 112
EPTP = 10080
NK = EPTP // K
NP = 10112
RPT = NP // NS

_MESH = plsc.VectorSubcoreMesh(
    core_axis_name="c", subcore_axis_name="s", num_cores=NC, num_subcores=NS)
_SC_PARAMS = pltpu.CompilerParams(needs_layout_passes=False, use_tc_tiling_on_sc=False)


def _pre_body(x_r, wg_r, asr_r, adr_r, wc_r, xw_r, as_r, ad_r, xg_r):
    xx = x_r[...]
    xw = jnp.dot(xx, wg_r[...], preferred_element_type=jnp.float32)
    xw_r[...] = xw
    as_r[...] = jnp.sum(xw * asr_r[...], axis=1, keepdims=True)
    ad_r[...] = jnp.sum(xw * adr_r[...], axis=1, keepdims=True)
    xg_r[...] = jnp.dot(xx, wc_r[...], preferred_element_type=jnp.float32)


def _tc_pre(x, W_gat, att_src, att_dst, W_gcn):
    return pl.pallas_call(
        _pre_body,
        out_shape=(
            jax.ShapeDtypeStruct((N, D), jnp.float32),
            jax.ShapeDtypeStruct((N, 1), jnp.float32),
            jax.ShapeDtypeStruct((N, 1), jnp.float32),
            jax.ShapeDtypeStruct((N, D), jnp.float32),
        ),
    )(x, W_gat, att_src.reshape(1, D), att_dst.reshape(1, D), W_gcn)


def _s1_body(src_h, dst_h, as_h, ad_h, p_h, denp_h, degp_h,
             asv, adv, den, deg, sidx, didx, pbuf):
    c = lax.axis_index("c")
    s = lax.axis_index("s")
    wid = c * NS + s
    base = wid * EPT
    pltpu.sync_copy(as_h, asv)
    pltpu.sync_copy(ad_h, adv)

    def zero(i, carry):
        den[pl.ds(i * L, L)] = jnp.zeros((L,), jnp.float32)
        deg[pl.ds(i * L, L)] = jnp.zeros((L,), jnp.float32)
        return carry

    lax.fori_loop(0, N // L, zero, 0)

    for cc in range(NC1):
        off = pl.multiple_of(base + cc * C1, 8)
        pltpu.sync_copy(src_h.at[pl.ds(off, C1)], sidx)
        pltpu.sync_copy(dst_h.at[pl.ds(off, C1)], didx)

        def work(j, carry):
            si = sidx[pl.ds(j * L, L)]
            di = didx[pl.ds(j * L, L)]
            av = plsc.load_gather(asv, [si])
            dv = plsc.load_gather(adv, [di])
            e = av + dv
            e = jnp.where(e > 0, e, 0.2 * e)
            p = jnp.exp(e)
            pbuf[pl.ds(j * L, L)] = p
            plsc.addupdate_scatter(den, [di], p)
            plsc.addupdate_scatter(deg, [di], jnp.full((L,), 1.0, jnp.float32))
            return carry

        lax.fori_loop(0, C1 // L, work, 0)
        pltpu.sync_copy(pbuf, p_h.at[pl.ds(off, C1)])

    pltpu.sync_copy(den, denp_h.at[pl.ds(wid * N, N)])
    pltpu.sync_copy(deg, degp_h.at[pl.ds(wid * N, N)])


def _sc_scalar(src, dst, a_s, a_d):
    return pl.kernel(
        _s1_body,
        out_type=(
            jax.ShapeDtypeStruct((E,), jnp.float32),
            jax.ShapeDtypeStruct((NW * N,), jnp.float32),
            jax.ShapeDtypeStruct((NW * N,), jnp.float32),
        ),
        mesh=_MESH,
        compiler_params=_SC_PARAMS,
        scratch_types=[
            pltpu.VMEM((N,), jnp.float32),
            pltpu.VMEM((N,), jnp.float32),
            pltpu.VMEM((N,), jnp.float32),
            pltpu.VMEM((N,), jnp.float32),
            pltpu.VMEM((C1,), jnp.int32),
            pltpu.VMEM((C1,), jnp.int32),
            pltpu.VMEM((C1,), jnp.float32),
        ],
    )(src, dst, a_s, a_d)


def _mid_body(denp_r, degp_r, as_r, ad_r, xg_r,
              xgd_r, den_r, dinv_r, ps_r):
    den_e = jnp.sum(denp_r[...], axis=1, keepdims=True)
    deg = jnp.sum(degp_r[...], axis=1, keepdims=True) + 1.0
    e = as_r[...] + ad_r[...]
    e = jnp.where(e > 0, e, 0.2 * e)
    ps = jnp.exp(e)
    ps_r[...] = ps
    den_r[...] = den_e + ps
    dinv = lax.rsqrt(deg)
    dinv_r[...] = dinv
    xgd_r[...] = xg_r[...] * dinv


def _tc_mid(denpT, degpT, a_s, a_d, xg):
    return pl.pallas_call(
        _mid_body,
        out_shape=(
            jax.ShapeDtypeStruct((N, D), jnp.float32),
            jax.ShapeDtypeStruct((N, 1), jnp.float32),
            jax.ShapeDtypeStruct((N, 1), jnp.float32),
            jax.ShapeDtypeStruct((N, 1), jnp.float32),
        ),
    )(denpT, degpT, a_s, a_d, xg)


def _s2_body(src_h, dst_h, p_h, xw_h, x_h, xgd_h, z_h,
             gat_h, gin_h, gcn_h,
             acc, sidxa, pca, didx0, didx1, rows0, rows1,
             sem0, sem1, msem0, msem1):
    c = lax.axis_index("c")
    s = lax.axis_index("s")
    wid = c * NS + s
    srow = s * RPT
    ebase = wid * EPTP
    sems = (sem0, sem1)
    bufs = (rows0, rows1)
    didxs = (didx0, didx1)
    msems = (msem0, msem1)

    pltpu.sync_copy(src_h.at[pl.ds(ebase, EPTP)], sidxa)
    pltpu.sync_copy(p_h.at[pl.ds(ebase, EPTP)], pca)

    def didx_start(cc, m):
        off = pl.multiple_of(ebase + cc * K, 8)
        pltpu.make_async_copy(dst_h.at[pl.ds(off, K)], didxs[m],
                              msems[m]).start()

    def didx_wait(cc, m):
        off = pl.multiple_of(ebase + cc * K, 8)
        pltpu.make_async_copy(dst_h.at[pl.ds(off, K)], didxs[m],
                              msems[m]).wait()

    def sidx_at(cc):
        return sidxa.at[pl.ds(pl.multiple_of(cc * K, 8), K)]

    for tab, out_h, scaled in ((xw_h, gat_h, True),
                               (x_h, gin_h, False),
                               (xgd_h, gcn_h, False)):
        pltpu.sync_copy(z_h, acc.at[pl.ds(srow, RPT)])
        plsc.subcore_barrier()

        didx_start(0, 0)
        didx_start(1, 1)
        pltpu.make_async_copy(tab.at[sidx_at(0)], bufs[0], sems[0]).start()

        def pair(i, carry):
            for b in range(2):
                cc = 2 * i + b
                nb = 1 - b
                rows = bufs[b]
                pltpu.make_async_copy(tab.at[sidx_at(cc)], rows,
                                      sems[b]).wait()

                @pl.when(cc + 1 < NK)
                def _():
                    pltpu.make_async_copy(tab.at[sidx_at(cc + 1)], bufs[nb],
                                          sems[nb]).start()
                if scaled:
                    def scale(e, cy):
                        psp = plsc.load_gather(
                            pca, [lax.broadcast(cc * K + e, (L,))])
                        for j in range(D // L):
                            rows[e, pl.ds(j * L, L)] = (
                                rows[e, pl.ds(j * L, L)] * psp)
                        return cy

                    lax.fori_loop(0, K, scale, 0, unroll=4)
                didx_wait(cc, b)
                pltpu.sync_copy(rows, acc.at[didxs[b]], add=True)

                @pl.when(cc + 2 < NK)
                def _():
                    didx_start(cc + 2, b)
            return carry

        lax.fori_loop(0, NK // 2, pair, 0)
        plsc.subcore_barrier()
        pltpu.sync_copy(acc.at[pl.ds(srow, RPT)], out_h.at[c, pl.ds(srow, RPT)])
        plsc.subcore_barrier()


def _sc_rows(src, dst, p, xw, x, xgd, zeros):
    return pl.kernel(
        _s2_body,
        out_type=(
            jax.ShapeDtypeStruct((NC, NP, D), jnp.float32),
            jax.ShapeDtypeStruct((NC, NP, D), jnp.float32),
            jax.ShapeDtypeStruct((NC, NP, D), jnp.float32),
        ),
        mesh=_MESH,
        compiler_params=_SC_PARAMS,
        scratch_types=[
            pltpu.VMEM_SHARED((NP, D), jnp.float32),
            pltpu.VMEM((EPTP,), jnp.int32),
            pltpu.VMEM((EPTP,), jnp.float32),
            pltpu.VMEM((K,), jnp.int32),
            pltpu.VMEM((K,), jnp.int32),
            pltpu.VMEM((K, D), jnp.float32),
            pltpu.VMEM((K, D), jnp.float32),
            pltpu.SemaphoreType.DMA,
            pltpu.SemaphoreType.DMA,
            pltpu.SemaphoreType.DMA,
            pltpu.SemaphoreType.DMA,
        ],
    )(src, dst, p, xw, x, xgd, zeros)


def _head1_body(gat_r, gin_r, gcn_r, xw_r, x_r, xg_r, ps_r, den_r, dinv_r,
                bg_r, w1_r, b1_r, w2_r, b2_r, bc_r,
                h_r, sums_r):
    ga = gat_r[0] + gat_r[1]
    xgat = (ga + ps_r[...] * xw_r[...]) / den_r[...] + bg_r[...]
    gi = gin_r[0] + gin_r[1]
    hh = x_r[...] + gi
    h1 = jnp.maximum(
        jnp.dot(hh, w1_r[...], preferred_element_type=jnp.float32) + b1_r[...],
        0.0)
    xgin = jnp.dot(h1, w2_r[...], preferred_element_type=jnp.float32) + b2_r[...]
    gc = gcn_r[0] + gcn_r[1]
    dv = dinv_r[...]
    xgcn = dv * gc + (dv * dv) * xg_r[...] + bc_r[...]
    hb = jnp.concatenate([xgat, xgin, xgcn], axis=1)
    h_r[...] = hb

    @pl.when(pl.program_id(0) == 0)
    def _():
        sums_r[...] = jnp.zeros_like(sums_r)

    s1 = jnp.sum(hb, axis=0, keepdims=True)
    s2 = jnp.sum(hb * hb, axis=0, keepdims=True)
    sums_r[...] += jnp.concatenate([s1, s2], axis=0)


def _tc_head1(gat, gin, gcn, xw, x, xg, ps, den, dinv,
              b_gat, gin_W1, gin_b1, gin_W2, gin_b2, b_gcn):
    B = 2000
    G = N // B
    full = lambda shape: pl.BlockSpec(shape, lambda i: (0,) * len(shape))
    row3 = pl.BlockSpec((NC, B, D), lambda i: (0, i, 0))
    row2 = pl.BlockSpec((B, D), lambda i: (i, 0))
    col = pl.BlockSpec((B, 1), lambda i: (i, 0))
    return pl.pallas_call(
        _head1_body,
        grid=(G,),
        in_specs=[
            row3, row3, row3, row2, row2, row2, col, col, col,
            full((1, D)), full((D, D)), full((1, D)), full((D, D)),
            full((1, D)), full((1, D)),
        ],
        out_specs=(
            pl.BlockSpec((B, 3 * D), lambda i: (i, 0)),
            pl.BlockSpec((2, 3 * D), lambda i: (0, 0)),
        ),
        out_shape=(
            jax.ShapeDtypeStruct((N, 3 * D), jnp.float32),
            jax.ShapeDtypeStruct((2, 3 * D), jnp.float32),
        ),
    )(gat, gin, gcn, xw, x, xg, ps, den, dinv,
      b_gat.reshape(1, D), gin_W1, gin_b1.reshape(1, D), gin_W2,
      gin_b2.reshape(1, D), b_gcn.reshape(1, D))


def _head2_body(h_r, sums_r, gam_r, bet_r, wl_r, bl_r, w2_r, b2_r, out_r):
    s = sums_r[...]
    mu = s[0:1] / float(N)
    var = s[1:2] / float(N) - mu * mu
    scale = gam_r[...] * lax.rsqrt(var + 1e-5)
    shift = bet_r[...] - mu * scale
    hb = jnp.maximum(h_r[...] * scale + shift, 0.0)
    h2 = jnp.maximum(
        jnp.dot(hb, wl_r[...], preferred_element_type=jnp.float32) + bl_r[...],
        0.0)
    out_r[...] = jnp.dot(h2, w2_r[...], preferred_element_type=jnp.float32) + b2_r[...]


def _tc_head2(h, sums, bn_gamma, bn_beta, W_lin, b_lin, W_lin2, b_lin2):
    B = 2000
    G = N // B
    full = lambda shape: pl.BlockSpec(shape, lambda i: (0,) * len(shape))
    return pl.pallas_call(
        _head2_body,
        grid=(G,),
        in_specs=[
            pl.BlockSpec((B, 3 * D), lambda i: (i, 0)),
            full((2, 3 * D)),
            full((1, 3 * D)), full((1, 3 * D)),
            full((3 * D, D)), full((1, D)),
            full((D, D)), full((1, D)),
        ],
        out_specs=pl.BlockSpec((B, D), lambda i: (i, 0)),
        out_shape=jax.ShapeDtypeStruct((N, D), jnp.float32),
    )(h, sums, bn_gamma.reshape(1, 3 * D), bn_beta.reshape(1, 3 * D),
      W_lin, b_lin.reshape(1, D), W_lin2, b_lin2.reshape(1, D))


def kernel(x, W_gat, att_src, att_dst, b_gat, gin_W1, gin_b1, gin_W2, gin_b2,
           W_gcn, b_gcn, bn_gamma, bn_beta, W_lin, b_lin, W_lin2, b_lin2,
           edge_index):
    src = edge_index[0]
    dst = edge_index[1]

    xw, a_s, a_d, xg = _tc_pre(x, W_gat, att_src, att_dst, W_gcn)

    p, denp, degp = _sc_scalar(src, dst,
                               a_s.reshape(N), a_d.reshape(N))

    xgd, den, dinv, ps = _tc_mid(denp.reshape(NW, N).T,
                                 degp.reshape(NW, N).T, a_s, a_d, xg)

    pad = ((0, 0), (0, EPTP - EPT))
    src_p = jnp.pad(src.reshape(NW, EPT), pad).reshape(NW * EPTP)
    dst_p = jnp.pad(dst.reshape(NW, EPT), pad,
                    constant_values=NP - 1).reshape(NW * EPTP)
    p_p = jnp.pad(p.reshape(NW, EPT), pad).reshape(NW * EPTP)
    zeros = jnp.zeros((RPT, D), jnp.float32)
    gat, gin, gcn = _sc_rows(src_p, dst_p, p_p, xw, x, xgd, zeros)

    h, sums = _tc_head1(gat, gin, gcn, xw, x, xg, ps, den, dinv,
                        b_gat, gin_W1, gin_b1, gin_W2, gin_b2, b_gcn)

    return _tc_head2(h, sums, bn_gamma, bn_beta, W_lin, b_lin, W_lin2, b_lin2)
